# Initial kernel scaffold; baseline (speedup 1.0000x reference)
import jax
import jax.numpy as jnp
from jax.experimental import pallas as pl


def kernel(x, w_gate, w_noise):
    raise NotImplementedError("write your pallas kernel here")


# speedup vs baseline: 4.7883x; 4.7883x over previous
import jax
import jax.numpy as jnp
from jax.experimental import pallas as pl
from jax.experimental.pallas import tpu as pltpu

_TOP_K = 8
_E = 64
_B = 8192
_D = 4096
_BM = 512
_NBLOCKS = _B // _BM


def _router_block_kernel(x_ref, w_ref, gates_ref, parts_ref):
    logits = jnp.dot(x_ref[...], w_ref[...], preferred_element_type=jnp.float32)
    m = jnp.max(logits, axis=1, keepdims=True)
    ex_full = jnp.exp(logits - m)
    lse = m[:, 0] + jnp.log(jnp.sum(ex_full, axis=1))

    iota = jax.lax.broadcasted_iota(jnp.int32, logits.shape, 1)
    work = logits
    mask = jnp.zeros(logits.shape, dtype=jnp.bool_)
    for _ in range(_TOP_K):
        mj = jnp.max(work, axis=1, keepdims=True)
        eq = work == mj
        idx = jnp.min(jnp.where(eq, iota, _E), axis=1, keepdims=True)
        hit = iota == idx
        mask = jnp.logical_or(mask, hit)
        work = jnp.where(hit, -jnp.inf, work)

    maskf = mask.astype(jnp.float32)
    exm = ex_full * maskf
    gates = exm / jnp.sum(exm, axis=1, keepdims=True)
    gates_ref[...] = gates

    imp = jnp.sum(gates, axis=0)
    load = jnp.sum(maskf, axis=0)
    zsum = jnp.sum(lse)
    rowi = jax.lax.broadcasted_iota(jnp.int32, (8, _E), 0)
    parts = (
        jnp.where(rowi == 0, imp[None, :], 0.0)
        + jnp.where(rowi == 1, load[None, :], 0.0)
        + jnp.where(rowi == 2, zsum, 0.0)
    )
    parts_ref[0, :, :] = parts


def _finalize_kernel(parts_ref, imp_ref, load_ref, loss_ref):
    total = jnp.sum(parts_ref[...], axis=0)
    imp = total[0:1, :]
    load = total[1:2, :]
    zsum = total[2, 0]

    def cv_sq(v):
        mean = jnp.sum(v) / _E
        var = jnp.sum((v - mean) ** 2) / (_E - 1)
        return var / (mean * mean + 1e-10)

    imp_ref[...] = imp
    load_ref[...] = load
    loss_ref[0, 0] = cv_sq(imp) + cv_sq(load) + zsum / _B


def kernel(x, w_gate, w_noise):
    del w_noise
    gates, parts = pl.pallas_call(
        _router_block_kernel,
        grid=(_NBLOCKS,),
        in_specs=[
            pl.BlockSpec((_BM, _D), lambda i: (i, 0)),
            pl.BlockSpec((_D, _E), lambda i: (0, 0)),
        ],
        out_specs=[
            pl.BlockSpec((_BM, _E), lambda i: (i, 0)),
            pl.BlockSpec((1, 8, _E), lambda i: (i, 0, 0)),
        ],
        out_shape=[
            jax.ShapeDtypeStruct((_B, _E), jnp.float32),
            jax.ShapeDtypeStruct((_NBLOCKS, 8, _E), jnp.float32),
        ],
        compiler_params=pltpu.CompilerParams(
            dimension_semantics=("parallel",),
        ),
    )(x, w_gate)

    imp, load, loss = pl.pallas_call(
        _finalize_kernel,
        in_specs=[pl.BlockSpec((_NBLOCKS, 8, _E), lambda: (0, 0, 0))],
        out_specs=[
            pl.BlockSpec((1, _E), lambda: (0, 0)),
            pl.BlockSpec((1, _E), lambda: (0, 0)),
            pl.BlockSpec(memory_space=pltpu.SMEM),
        ],
        out_shape=[
            jax.ShapeDtypeStruct((1, _E), jnp.float32),
            jax.ShapeDtypeStruct((1, _E), jnp.float32),
            jax.ShapeDtypeStruct((1, 1), jnp.float32),
        ],
    )(parts)

    return gates, loss[0, 0], imp[0], load[0]


# speedup vs baseline: 5.7779x; 1.2067x over previous
import jax
import jax.numpy as jnp
from jax.experimental import pallas as pl
from jax.experimental.pallas import tpu as pltpu

_TOP_K = 8
_E = 64
_B = 8192
_D = 4096
_BM = 512
_NBLOCKS = _B // _BM


def _router_block_kernel(x_ref, w_ref, gates_ref, parts_ref):
    logits = jnp.dot(x_ref[...], w_ref[...], preferred_element_type=jnp.float32)

    work = logits
    m = None
    t = None
    for r in range(_TOP_K):
        t = jnp.max(work, axis=1, keepdims=True)
        if r == 0:
            m = t
        if r != _TOP_K - 1:
            work = jnp.where(work == t, -jnp.inf, work)
    mask = logits >= t

    ex_full = jnp.exp(logits - m)
    lse = m[:, 0] + jnp.log(jnp.sum(ex_full, axis=1))

    exm = jnp.where(mask, ex_full, 0.0)
    gates = exm / jnp.sum(exm, axis=1, keepdims=True)
    gates_ref[...] = gates

    imp = jnp.sum(gates, axis=0)
    load = jnp.sum(mask.astype(jnp.float32), axis=0)
    zsum = jnp.sum(lse)
    rowi = jax.lax.broadcasted_iota(jnp.int32, (8, _E), 0)
    parts = (
        jnp.where(rowi == 0, imp[None, :], 0.0)
        + jnp.where(rowi == 1, load[None, :], 0.0)
        + jnp.where(rowi == 2, zsum, 0.0)
    )
    parts_ref[0, :, :] = parts


def _finalize_kernel(parts_ref, imp_ref, load_ref, loss_ref):
    total = jnp.sum(parts_ref[...], axis=0)
    imp = total[0:1, :]
    load = total[1:2, :]
    zsum = total[2, 0]

    def cv_sq(v):
        mean = jnp.sum(v) / _E
        var = jnp.sum((v - mean) ** 2) / (_E - 1)
        return var / (mean * mean + 1e-10)

    imp_ref[...] = imp
    load_ref[...] = load
    loss_ref[0, 0] = cv_sq(imp) + cv_sq(load) + zsum / _B


def kernel(x, w_gate, w_noise):
    del w_noise
    gates, parts = pl.pallas_call(
        _router_block_kernel,
        grid=(_NBLOCKS,),
        in_specs=[
            pl.BlockSpec((_BM, _D), lambda i: (i, 0)),
            pl.BlockSpec((_D, _E), lambda i: (0, 0)),
        ],
        out_specs=[
            pl.BlockSpec((_BM, _E), lambda i: (i, 0)),
            pl.BlockSpec((1, 8, _E), lambda i: (i, 0, 0)),
        ],
        out_shape=[
            jax.ShapeDtypeStruct((_B, _E), jnp.float32),
            jax.ShapeDtypeStruct((_NBLOCKS, 8, _E), jnp.float32),
        ],
        compiler_params=pltpu.CompilerParams(
            dimension_semantics=("parallel",),
        ),
    )(x, w_gate)

    imp, load, loss = pl.pallas_call(
        _finalize_kernel,
        in_specs=[pl.BlockSpec((_NBLOCKS, 8, _E), lambda: (0, 0, 0))],
        out_specs=[
            pl.BlockSpec((1, _E), lambda: (0, 0)),
            pl.BlockSpec((1, _E), lambda: (0, 0)),
            pl.BlockSpec(memory_space=pltpu.SMEM),
        ],
        out_shape=[
            jax.ShapeDtypeStruct((1, _E), jnp.float32),
            jax.ShapeDtypeStruct((1, _E), jnp.float32),
            jax.ShapeDtypeStruct((1, 1), jnp.float32),
        ],
    )(parts)

    return gates, loss[0, 0], imp[0], load[0]


# speedup vs baseline: 6.1146x; 1.0583x over previous
import jax
import jax.numpy as jnp
from jax.experimental import pallas as pl
from jax.experimental.pallas import tpu as pltpu

_TOP_K = 8
_E = 64
_B = 8192
_D = 4096
_BM = 1024
_NBLOCKS = _B // _BM


def _router_block_kernel(x_ref, w_ref, gates_ref, parts_ref):
    logits = jnp.dot(x_ref[...], w_ref[...], preferred_element_type=jnp.float32)

    work = logits
    m = None
    t = None
    for r in range(_TOP_K):
        t = jnp.max(work, axis=1, keepdims=True)
        if r == 0:
            m = t
        if r != _TOP_K - 1:
            work = jnp.where(work == t, -jnp.inf, work)
    mask = logits >= t

    ex_full = jnp.exp(logits - m)
    lse = m[:, 0] + jnp.log(jnp.sum(ex_full, axis=1))

    exm = jnp.where(mask, ex_full, 0.0)
    gates = exm / jnp.sum(exm, axis=1, keepdims=True)
    gates_ref[...] = gates

    imp = jnp.sum(gates, axis=0)
    load = jnp.sum(mask.astype(jnp.float32), axis=0)
    zsum = jnp.sum(lse)
    rowi = jax.lax.broadcasted_iota(jnp.int32, (8, _E), 0)
    parts = (
        jnp.where(rowi == 0, imp[None, :], 0.0)
        + jnp.where(rowi == 1, load[None, :], 0.0)
        + jnp.where(rowi == 2, zsum, 0.0)
    )
    parts_ref[0, :, :] = parts


def _finalize_kernel(parts_ref, imp_ref, load_ref, loss_ref):
    total = jnp.sum(parts_ref[...], axis=0)
    imp = total[0:1, :]
    load = total[1:2, :]
    zsum = total[2, 0]

    def cv_sq(v):
        mean = jnp.sum(v) / _E
        var = jnp.sum((v - mean) ** 2) / (_E - 1)
        return var / (mean * mean + 1e-10)

    imp_ref[...] = imp
    load_ref[...] = load
    loss_ref[0, 0] = cv_sq(imp) + cv_sq(load) + zsum / _B


def kernel(x, w_gate, w_noise):
    del w_noise
    gates, parts = pl.pallas_call(
        _router_block_kernel,
        grid=(_NBLOCKS,),
        in_specs=[
            pl.BlockSpec((_BM, _D), lambda i: (i, 0)),
            pl.BlockSpec((_D, _E), lambda i: (0, 0)),
        ],
        out_specs=[
            pl.BlockSpec((_BM, _E), lambda i: (i, 0)),
            pl.BlockSpec((1, 8, _E), lambda i: (i, 0, 0)),
        ],
        out_shape=[
            jax.ShapeDtypeStruct((_B, _E), jnp.float32),
            jax.ShapeDtypeStruct((_NBLOCKS, 8, _E), jnp.float32),
        ],
        compiler_params=pltpu.CompilerParams(
            dimension_semantics=("parallel",),
        ),
    )(x, w_gate)

    imp, load, loss = pl.pallas_call(
        _finalize_kernel,
        in_specs=[pl.BlockSpec((_NBLOCKS, 8, _E), lambda: (0, 0, 0))],
        out_specs=[
            pl.BlockSpec((1, _E), lambda: (0, 0)),
            pl.BlockSpec((1, _E), lambda: (0, 0)),
            pl.BlockSpec(memory_space=pltpu.SMEM),
        ],
        out_shape=[
            jax.ShapeDtypeStruct((1, _E), jnp.float32),
            jax.ShapeDtypeStruct((1, _E), jnp.float32),
            jax.ShapeDtypeStruct((1, 1), jnp.float32),
        ],
    )(parts)

    return gates, loss[0, 0], imp[0], load[0]
